# baseline (speedup 1.0000x reference)
import jax
import jax.numpy as jnp
from jax.experimental import pallas as pl


def _bcast_body(emb_ref, out_ref):
    out_ref[...] = jnp.broadcast_to(emb_ref[...], out_ref.shape)


def kernel(sequence, embeddings):
    batch, seq_len, feat = sequence.shape
    emb = jax.lax.slice(embeddings, (0, 0), (seq_len, feat))

    blk = 512
    while seq_len % blk != 0:
        blk //= 2
    grid = (seq_len // blk,)

    return pl.pallas_call(
        _bcast_body,
        grid=grid,
        in_specs=[pl.BlockSpec((blk, feat), lambda s: (s, 0))],
        out_specs=pl.BlockSpec((batch, blk, feat), lambda s: (0, s, 0)),
        out_shape=jax.ShapeDtypeStruct((batch, seq_len, feat), sequence.dtype),
    )(emb)


# speedup vs baseline: 1.0433x; 1.0433x over previous
import jax
import jax.numpy as jnp
from jax.experimental import pallas as pl


def _bcast_body(emb_ref, out_ref):
    out_ref[...] = jnp.broadcast_to(emb_ref[...], out_ref.shape)


def kernel(sequence, embeddings):
    batch, seq_len, feat = sequence.shape

    blk = 1024
    while seq_len % blk != 0:
        blk //= 2
    grid = (seq_len // blk,)

    return pl.pallas_call(
        _bcast_body,
        grid=grid,
        in_specs=[pl.BlockSpec((blk, feat), lambda s: (s, 0))],
        out_specs=pl.BlockSpec((batch, blk, feat), lambda s: (0, s, 0)),
        out_shape=jax.ShapeDtypeStruct((batch, seq_len, feat), sequence.dtype),
    )(embeddings)
